# baseline (speedup 1.0000x reference)
import jax
import jax.numpy as jnp
from jax.experimental import pallas as pl
from jax.experimental.pallas import tpu as pltpu

L = 2
EPS = 0.0
AGGRS = ((1, 1, 1), (1, 1, 2), (1, 2, 2), (2, 1, 1), (2, 1, 2), (2, 2, 2))


def _bn(x, g, b):
    mu = jnp.mean(x, axis=0, keepdims=True)
    var = jnp.var(x, axis=0, keepdims=True)
    return (x - mu) / jnp.sqrt(var + 1e-5) * g + b


def _mm_kernel(x_ref, w_ref, b_ref, o_ref):
    o_ref[...] = jnp.dot(x_ref[...], w_ref[...],
                         preferred_element_type=jnp.float32) + b_ref[...]


def _mm(x, w, b):
    R, C = x.shape
    BLK = 400
    grid = (R + BLK - 1) // BLK
    return pl.pallas_call(
        _mm_kernel,
        grid=(grid,),
        in_specs=[pl.BlockSpec((BLK, C), lambda i: (i, 0)),
                  pl.BlockSpec((C, C), lambda i: (0, 0)),
                  pl.BlockSpec((C,), lambda i: (0,))],
        out_specs=pl.BlockSpec((BLK, C), lambda i: (i, 0)),
        out_shape=jax.ShapeDtypeStruct((R, C), jnp.float32),
    )(x, w, b)


def kernel(a0, a1, a2, ei1, ei2, tri_111, tri_112, tri_122, tri_211, tri_212, tri_222, inv1, inv2, W_gnn, b_gnn, gamma, beta, W_out, b_out):
    tris = {(1, 1, 1): tri_111, (1, 1, 2): tri_112, (1, 2, 2): tri_122,
            (2, 1, 1): tri_211, (2, 1, 2): tri_212, (2, 2, 2): tri_222}
    eis = [None, ei1, ei2]
    invs = [None, inv1, inv2]
    attrs = [a0, a1, a2]
    for layer in range(L):
        aggs = [jnp.zeros_like(a) for a in attrs]
        for l in (1, 2):
            ei = eis[l]
            aggs[0] = aggs[0].at[ei[0]].add(attrs[l]).at[ei[1]].add(attrs[l])
            aggs[l] = aggs[l] + attrs[0][ei[0]] * attrs[0][ei[1]]
        for (i, j, k), tri in tris.items():
            msg = attrs[j][tri[1]] * attrs[k][tri[2]]
            aggs[i] = aggs[i].at[tri[0]].add(msg)
        new_attrs = []
        for l in range(3):
            h = (1.0 + EPS) * attrs[l] + aggs[l]
            if l > 0:
                h = 0.5 * (h + h[invs[l]])
            h = h @ W_gnn[layer, l] + b_gnn[layer, l]
            h = jax.nn.relu(_bn(h, gamma[layer, l], beta[layer, l]))
            new_attrs.append(h)
        attrs = new_attrs
    out = tuple(_mm(attrs[l], W_out[l], b_out[l]) for l in range(3))
    return out
